# baseline (speedup 1.0000x reference)
import functools

import jax
from jax import lax as _lax
import jax.numpy as jnp
from jax import lax
from jax.experimental import pallas as pl
from jax.experimental.pallas import tpu as pltpu
from jax.experimental.pallas import tpu_sc as plsc

NN = 10000
EE = 320000
DFEAT = 128
NH = 4
DH = 32

NC = 2
NS = 16
NW = NC * NS
BLK = 128
NEG = -1e38

_ROWS_PER_BLOCK = 1000

_GATHER_DNUMS = _lax.GatherDimensionNumbers(
    offset_dims=(), collapsed_slice_dims=(0,), start_index_map=(0,))


def _lane_allreduce_sum(v):
    for s in (1, 2, 4, 8):
        idx = _lax.iota(jnp.int32, 16) ^ s
        v = v + _lax.gather(v, idx[:, None], _GATHER_DNUMS, slice_sizes=(1,),
                            mode=_lax.GatherScatterMode.PROMISE_IN_BOUNDS)
    return v


def _proj_body(nodes_ref, wl_ref, bl_ref, wr_ref, br_ref, ql_ref, qr_ref):
    x = nodes_ref[...]
    ql_ref[...] = jnp.dot(x, wl_ref[...], preferred_element_type=jnp.float32) + bl_ref[...]
    qr_ref[...] = jnp.dot(x, wr_ref[...], preferred_element_type=jnp.float32) + br_ref[...]


def _project(nodes, W_l, b_l, W_r, b_r):
    n = nodes.shape[0]
    k = W_l.shape[1]
    grid = n // _ROWS_PER_BLOCK
    return pl.pallas_call(
        _proj_body,
        grid=(grid,),
        in_specs=[
            pl.BlockSpec((_ROWS_PER_BLOCK, DFEAT), lambda i: (i, 0)),
            pl.BlockSpec((DFEAT, k), lambda i: (0, 0)),
            pl.BlockSpec((1, k), lambda i: (0, 0)),
            pl.BlockSpec((DFEAT, k), lambda i: (0, 0)),
            pl.BlockSpec((1, k), lambda i: (0, 0)),
        ],
        out_specs=[
            pl.BlockSpec((_ROWS_PER_BLOCK, k), lambda i: (i, 0)),
            pl.BlockSpec((_ROWS_PER_BLOCK, k), lambda i: (i, 0)),
        ],
        out_shape=[
            jax.ShapeDtypeStruct((n, k), jnp.float32),
            jax.ShapeDtypeStruct((n, k), jnp.float32),
        ],
    )(nodes, W_l, b_l.reshape(1, k), W_r, b_r.reshape(1, k))


def _edge_body(ql_h, qr_h, snd_h, rcv_h, nb_h, eb_h, wa_h, out_h,
               sidx, ridx, rsc, sent, recv, rowbuf, zrow, nb_v, eb_v, wa_v,
               sem_s, sem_r):
    wid = lax.axis_index("c") * NS + lax.axis_index("s")
    pltpu.sync_copy(nb_h, nb_v)
    pltpu.sync_copy(eb_h, eb_v)
    pltpu.sync_copy(wa_h, wa_v)
    nbv = nb_v[pl.ds(wid, 16)]
    ebv = eb_v[pl.ds(wid, 16)]
    n_lo = nbv[0]
    n_hi = nbv[1]
    e0 = ebv[0]
    e1 = ebv[1]
    wa = [wa_v[pl.ds(16 * j, 16)] for j in range(2)]

    zv = jnp.zeros((16,), jnp.float32)
    for j in range(8):
        zrow[pl.ds(16 * j, 16)] = zv

    def write_row(node, dvs, avs):
        for j in range(8):
            rowbuf[pl.ds(16 * j, 16)] = avs[j] / dvs[j // 2]
        pltpu.sync_copy(rowbuf, out_h.at[node])

    def zero_rows(lo, hi):
        def zbody(g, c):
            pltpu.sync_copy(zrow, out_h.at[g])
            return c
        lax.fori_loop(lo, hi, zbody, 0)

    base0 = (e0 // BLK) * BLK
    nblk = (e1 - base0 + BLK - 1) // BLK

    negv = jnp.full((16,), NEG, jnp.float32)

    def stage_block(b, par):
        bstart = base0 + b * BLK
        pltpu.sync_copy(snd_h.at[pl.ds(bstart, BLK)], sidx.at[par])
        pltpu.sync_copy(rcv_h.at[pl.ds(bstart, BLK)], ridx.at[par])
        pltpu.sync_copy(rcv_h.at[pl.ds(bstart, BLK)], rsc.at[par, pl.ds(0, BLK)])
        pltpu.async_copy(ql_h.at[sidx.at[par]], sent.at[par], sem_s.at[par])
        pltpu.async_copy(qr_h.at[ridx.at[par]], recv.at[par], sem_r.at[par])

    @pl.when(nblk > 0)
    def _():
        stage_block(jnp.int32(0), jnp.int32(0))

    def blk_body(blk, carry):
        bstart = base0 + blk * BLK
        par = lax.rem(blk, 2)

        @pl.when(blk + 1 < nblk)
        def _():
            stage_block(blk + 1, 1 - par)

        pltpu.make_async_copy(ql_h.at[sidx.at[par]], sent.at[par],
                              sem_s.at[par]).wait()
        pltpu.make_async_copy(qr_h.at[ridx.at[par]], recv.at[par],
                              sem_r.at[par]).wait()
        lo = jnp.maximum(e0, bstart)
        hi = jnp.minimum(e1, bstart + BLK)

        def edge_body(e, ec):
            cur = ec[0]
            ms = list(ec[1:5])
            ds = list(ec[5:9])
            avs = list(ec[9:17])
            i = e - bstart
            r = rsc[par, pl.ds(i, 16)][0]
            is_new = r != cur

            @pl.when(is_new)
            def _():
                @pl.when(cur >= n_lo)
                def _():
                    write_row(cur, ds, avs)
                zero_rows(cur + 1, r)

            svs = [sent[par, i, pl.ds(16 * j, 16)] for j in range(8)]
            rvs = [recv[par, i, pl.ds(16 * j, 16)] for j in range(8)]
            pvs = []
            for j in range(8):
                z = svs[j] + rvs[j]
                z = jnp.maximum(z, 0.2 * z)
                pvs.append(z * wa[j % 2])
            ones = jnp.ones((16,), jnp.float32)
            for h in range(4):
                lv = _lane_allreduce_sum(pvs[2 * h] + pvs[2 * h + 1])
                msel = jnp.where(is_new, negv, ms[h])
                diff = lv - msel
                u = jnp.exp(-jnp.abs(diff))
                pos = diff > 0.0
                sc = jnp.where(pos, u, ones)
                el = jnp.where(pos, ones, u)
                ds[h] = ds[h] * sc + el
                avs[2 * h] = avs[2 * h] * sc + el * svs[2 * h]
                avs[2 * h + 1] = avs[2 * h + 1] * sc + el * svs[2 * h + 1]
                ms[h] = jnp.maximum(msel, lv)
            return (r, *ms, *ds, *avs)

        return lax.fori_loop(lo, hi, edge_body, carry)

    zvec = jnp.zeros((16,), jnp.float32)
    init = (n_lo - 1,
            negv, negv, negv, negv,
            zvec, zvec, zvec, zvec,
            zvec, zvec, zvec, zvec, zvec, zvec, zvec, zvec)
    fin = lax.fori_loop(0, nblk, blk_body, init)
    cur = fin[0]
    ds = list(fin[5:9])
    avs = list(fin[9:17])

    @pl.when(cur >= n_lo)
    def _():
        write_row(cur, ds, avs)

    zero_rows(cur + 1, n_hi)


@functools.partial(
    pl.kernel,
    out_type=jax.ShapeDtypeStruct((NN, DFEAT), jnp.float32),
    mesh=plsc.VectorSubcoreMesh(core_axis_name="c", subcore_axis_name="s",
                                num_cores=NC, num_subcores=NS),
    scratch_types=[
        pltpu.VMEM((2, BLK), jnp.int32),
        pltpu.VMEM((2, BLK), jnp.int32),
        pltpu.VMEM((2, BLK + 16), jnp.int32),
        pltpu.VMEM((2, BLK, DFEAT), jnp.float32),
        pltpu.VMEM((2, BLK, DFEAT), jnp.float32),
        pltpu.VMEM((DFEAT,), jnp.float32),
        pltpu.VMEM((DFEAT,), jnp.float32),
        pltpu.VMEM((48,), jnp.int32),
        pltpu.VMEM((48,), jnp.int32),
        pltpu.VMEM((32,), jnp.float32),
        pltpu.SemaphoreType.DMA((2,)),
        pltpu.SemaphoreType.DMA((2,)),
    ],
    compiler_params=pltpu.CompilerParams(needs_layout_passes=False),
)
def _edge_kernel(*refs):
    _edge_body(*refs)


def kernel(nodes, senders, receivers, W_l, b_l, W_r, b_r, W_a, b_a):
    ql, qr = _project(nodes, W_l, b_l, W_r, b_r)
    pos = jnp.arange(1, NW, dtype=jnp.int32) * (EE // NW)
    nb_mid = receivers[pos]
    node_bounds = jnp.concatenate([
        jnp.zeros((1,), jnp.int32), nb_mid,
        jnp.full((1,), NN, jnp.int32),
    ])
    edge_bounds = jnp.searchsorted(receivers, node_bounds).astype(jnp.int32)
    nb_pad = jnp.zeros((48,), jnp.int32).at[:NW + 1].set(node_bounds)
    eb_pad = jnp.zeros((48,), jnp.int32).at[:NW + 1].set(edge_bounds)
    wa = W_a.reshape(DH)
    out = _edge_kernel(ql, qr, senders, receivers, nb_pad, eb_pad, wa)
    return out


# speedup vs baseline: 1.7843x; 1.7843x over previous
import functools

import jax
from jax import lax as _lax
import jax.numpy as jnp
from jax import lax
from jax.experimental import pallas as pl
from jax.experimental.pallas import tpu as pltpu
from jax.experimental.pallas import tpu_sc as plsc

NN = 10000
EE = 320000
DFEAT = 128
NH = 4
DH = 32

NC = 2
NS = 16
NW = NC * NS
BLK = 128
NEG = -1e38

_ROWS_PER_BLOCK = 1000

_GATHER_DNUMS = _lax.GatherDimensionNumbers(
    offset_dims=(), collapsed_slice_dims=(0,), start_index_map=(0,))


def _lane_perm(v, idx):
    return _lax.gather(v, idx[:, None], _GATHER_DNUMS, slice_sizes=(1,),
                       mode=_lax.GatherScatterMode.PROMISE_IN_BOUNDS)


def _lane_allreduce_sum(v):
    for s in (1, 2, 4, 8):
        idx = _lax.iota(jnp.int32, 16) ^ s
        v = v + _lax.gather(v, idx[:, None], _GATHER_DNUMS, slice_sizes=(1,),
                            mode=_lax.GatherScatterMode.PROMISE_IN_BOUNDS)
    return v


def _proj_body(nodes_ref, wl_ref, bl_ref, wr_ref, br_ref, ql_ref, qr_ref):
    x = nodes_ref[...]
    ql_ref[...] = jnp.dot(x, wl_ref[...], preferred_element_type=jnp.float32) + bl_ref[...]
    qr_ref[...] = jnp.dot(x, wr_ref[...], preferred_element_type=jnp.float32) + br_ref[...]


def _project(nodes, W_l, b_l, W_r, b_r):
    n = nodes.shape[0]
    k = W_l.shape[1]
    grid = n // _ROWS_PER_BLOCK
    return pl.pallas_call(
        _proj_body,
        grid=(grid,),
        in_specs=[
            pl.BlockSpec((_ROWS_PER_BLOCK, DFEAT), lambda i: (i, 0)),
            pl.BlockSpec((DFEAT, k), lambda i: (0, 0)),
            pl.BlockSpec((1, k), lambda i: (0, 0)),
            pl.BlockSpec((DFEAT, k), lambda i: (0, 0)),
            pl.BlockSpec((1, k), lambda i: (0, 0)),
        ],
        out_specs=[
            pl.BlockSpec((_ROWS_PER_BLOCK, k), lambda i: (i, 0)),
            pl.BlockSpec((_ROWS_PER_BLOCK, k), lambda i: (i, 0)),
        ],
        out_shape=[
            jax.ShapeDtypeStruct((n, k), jnp.float32),
            jax.ShapeDtypeStruct((n, k), jnp.float32),
        ],
    )(nodes, W_l, b_l.reshape(1, k), W_r, b_r.reshape(1, k))


def _edge_body(ql_h, qr_h, snd_h, rcv_h, nb_h, eb_h, wa_h, out_h,
               sidx, ridx, rsc, sent, recv, rowbuf, zrow, nb_v, eb_v, wa_v,
               bpos, bval, sem_s, sem_r):
    wid = lax.axis_index("c") * NS + lax.axis_index("s")
    pltpu.sync_copy(nb_h, nb_v)
    pltpu.sync_copy(eb_h, eb_v)
    pltpu.sync_copy(wa_h, wa_v)
    nbv = nb_v[pl.ds(wid, 16)]
    ebv = eb_v[pl.ds(wid, 16)]
    n_lo = nbv[0]
    n_hi = nbv[1]
    e0 = ebv[0]
    e1 = ebv[1]
    wa = [wa_v[pl.ds(16 * j, 16)] for j in range(2)]

    zv = jnp.zeros((16,), jnp.float32)
    for j in range(8):
        zrow[pl.ds(16 * j, 16)] = zv

    def write_row(node, dvs, avs):
        for j in range(8):
            rowbuf[pl.ds(16 * j, 16)] = avs[j] / dvs[j // 2]
        pltpu.sync_copy(rowbuf, out_h.at[node])

    def zero_rows(lo, hi):
        def zbody(g, c):
            pltpu.sync_copy(zrow, out_h.at[g])
            return c
        lax.fori_loop(lo, hi, zbody, 0)

    base0 = (e0 // BLK) * BLK
    nblk = (e1 - base0 + BLK - 1) // BLK

    negv = jnp.full((16,), NEG, jnp.float32)

    def stage_block(b, par):
        bstart = base0 + b * BLK
        pltpu.sync_copy(snd_h.at[pl.ds(bstart, BLK)], sidx.at[par])
        pltpu.sync_copy(rcv_h.at[pl.ds(bstart, BLK)], ridx.at[par])

        rbase = par * (BLK + 16)

        @pl.when(bstart == 0)
        def _():
            rsc[pl.ds(rbase, 16)] = jnp.full((16,), -1, jnp.int32)
            pltpu.sync_copy(rcv_h.at[pl.ds(0, BLK)], rsc.at[pl.ds(rbase + 8, BLK)])

        @pl.when(bstart != 0)
        def _():
            pltpu.sync_copy(rcv_h.at[pl.ds(bstart - 8, BLK + 8)],
                            rsc.at[pl.ds(rbase, BLK + 8)])

        pltpu.async_copy(ql_h.at[sidx.at[par]], sent.at[par], sem_s.at[par])
        pltpu.async_copy(qr_h.at[ridx.at[par]], recv.at[par], sem_r.at[par])

    @pl.when(nblk > 0)
    def _():
        stage_block(jnp.int32(0), jnp.int32(0))

    ones = jnp.ones((16,), jnp.float32)
    iota16 = _lax.iota(jnp.int32, 16)

    def blk_body(blk, carry):
        bstart = base0 + blk * BLK
        par = lax.rem(blk, 2)

        @pl.when(blk + 1 < nblk)
        def _():
            stage_block(blk + 1, 1 - par)

        pltpu.make_async_copy(ql_h.at[sidx.at[par]], sent.at[par],
                              sem_s.at[par]).wait()
        pltpu.make_async_copy(qr_h.at[ridx.at[par]], recv.at[par],
                              sem_r.at[par]).wait()
        lo = jnp.maximum(e0, bstart)
        hi = jnp.minimum(e1, bstart + BLK)

        cnt_vec = jnp.zeros((16,), jnp.int32)
        shift_idx = jnp.maximum(iota16 - 1, 0)
        splat15 = jnp.full((16,), 15, jnp.int32)
        rbase = par * (BLK + 16)
        lastv = _lane_perm(rsc[pl.ds(rbase, 16)], jnp.full((16,), 7, jnp.int32))
        for g in range(BLK // 16):
            curv = rsc[pl.ds(rbase + 8 + 16 * g, 16)]
            prevv = jnp.where(iota16 == 0, lastv, _lane_perm(curv, shift_idx))
            lastv = _lane_perm(curv, splat15)
            posv = bstart + 16 * g + iota16
            mask = (curv != prevv) & (posv >= lo) & (posv < hi)
            incl = plsc.cumsum(mask.astype(jnp.int32))
            idxv = cnt_vec + incl - 1
            plsc.store_scatter(bpos, [idxv], posv, mask=mask)
            plsc.store_scatter(bval, [idxv], curv, mask=mask)
            cnt_vec = cnt_vec + _lane_perm(incl, splat15)
        plsc.store_scatter(bpos, [cnt_vec], jnp.full((16,), 1, jnp.int32) * hi,
                           mask=iota16 == 0)
        cnt = cnt_vec[0]

        def seg_body(s, sc_carry):
            cur = sc_carry[0]
            start_e = sc_carry[1]
            sv = jnp.full((16,), 1, jnp.int32) * s
            end_e = plsc.load_gather(bpos, [sv])[0]
            i0 = jnp.clip(start_e - bstart, 0, BLK - 1)
            rvs = [recv[par, i0, pl.ds(16 * j, 16)] for j in range(8)]

            def edge_body(e, ec):
                ms = list(ec[0:4])
                ds = list(ec[4:8])
                avs = list(ec[8:16])
                i = e - bstart
                svs = [sent[par, i, pl.ds(16 * j, 16)] for j in range(8)]
                pvs = []
                for j in range(8):
                    z = svs[j] + rvs[j]
                    z = jnp.maximum(z, 0.2 * z)
                    pvs.append(z * wa[j % 2])
                for h in range(4):
                    lv = _lane_allreduce_sum(pvs[2 * h] + pvs[2 * h + 1])
                    diff = lv - ms[h]
                    u = jnp.exp(-jnp.abs(diff))
                    pos = diff > 0.0
                    sc = jnp.where(pos, u, ones)
                    el = jnp.where(pos, ones, u)
                    ds[h] = ds[h] * sc + el
                    avs[2 * h] = avs[2 * h] * sc + el * svs[2 * h]
                    avs[2 * h + 1] = avs[2 * h + 1] * sc + el * svs[2 * h + 1]
                    ms[h] = jnp.maximum(ms[h], lv)
                return (*ms, *ds, *avs)

            ec = lax.fori_loop(start_e, end_e, edge_body, tuple(sc_carry[2:]))
            ms = list(ec[0:4])
            ds = list(ec[4:8])
            avs = list(ec[8:16])

            fin_seg = s < cnt
            bv = plsc.load_gather(bval, [jnp.minimum(sv, 127)])[0]

            @pl.when(fin_seg)
            def _():
                @pl.when(cur >= n_lo)
                def _():
                    write_row(cur, ds, avs)
                zero_rows(cur + 1, bv)

            cur2 = jnp.where(fin_seg, bv, cur)
            ms = [jnp.where(fin_seg, negv, m) for m in ms]
            zv16 = jnp.zeros((16,), jnp.float32)
            ds = [jnp.where(fin_seg, zv16, d) for d in ds]
            avs = [jnp.where(fin_seg, zv16, a) for a in avs]
            return (cur2, end_e, *ms, *ds, *avs)

        return lax.fori_loop(0, cnt + 1, seg_body, carry)

    zvec = jnp.zeros((16,), jnp.float32)
    init = (n_lo - 1, e0,
            negv, negv, negv, negv,
            zvec, zvec, zvec, zvec,
            zvec, zvec, zvec, zvec, zvec, zvec, zvec, zvec)
    fin = lax.fori_loop(0, nblk, blk_body, init)
    cur = fin[0]
    ds = list(fin[6:10])
    avs = list(fin[10:18])

    @pl.when(cur >= n_lo)
    def _():
        write_row(cur, ds, avs)

    zero_rows(cur + 1, n_hi)


@functools.partial(
    pl.kernel,
    out_type=jax.ShapeDtypeStruct((NN, DFEAT), jnp.float32),
    mesh=plsc.VectorSubcoreMesh(core_axis_name="c", subcore_axis_name="s",
                                num_cores=NC, num_subcores=NS),
    scratch_types=[
        pltpu.VMEM((2, BLK), jnp.int32),
        pltpu.VMEM((2, BLK), jnp.int32),
        pltpu.VMEM((2 * (BLK + 16),), jnp.int32),
        pltpu.VMEM((2, BLK, DFEAT), jnp.float32),
        pltpu.VMEM((2, BLK, DFEAT), jnp.float32),
        pltpu.VMEM((DFEAT,), jnp.float32),
        pltpu.VMEM((DFEAT,), jnp.float32),
        pltpu.VMEM((48,), jnp.int32),
        pltpu.VMEM((48,), jnp.int32),
        pltpu.VMEM((32,), jnp.float32),
        pltpu.VMEM((160,), jnp.int32),
        pltpu.VMEM((160,), jnp.int32),
        pltpu.SemaphoreType.DMA((2,)),
        pltpu.SemaphoreType.DMA((2,)),
    ],
    compiler_params=pltpu.CompilerParams(needs_layout_passes=False),
)
def _edge_kernel(*refs):
    _edge_body(*refs)


def kernel(nodes, senders, receivers, W_l, b_l, W_r, b_r, W_a, b_a):
    ql, qr = _project(nodes, W_l, b_l, W_r, b_r)
    pos = jnp.arange(1, NW, dtype=jnp.int32) * (EE // NW)
    nb_mid = receivers[pos]
    node_bounds = jnp.concatenate([
        jnp.zeros((1,), jnp.int32), nb_mid,
        jnp.full((1,), NN, jnp.int32),
    ])
    edge_bounds = jnp.searchsorted(receivers, node_bounds).astype(jnp.int32)
    nb_pad = jnp.zeros((48,), jnp.int32).at[:NW + 1].set(node_bounds)
    eb_pad = jnp.zeros((48,), jnp.int32).at[:NW + 1].set(edge_bounds)
    wa = W_a.reshape(DH)
    out = _edge_kernel(ql, qr, senders, receivers, nb_pad, eb_pad, wa)
    return out
